# baseline (speedup 1.0000x reference)
import functools

import jax
import jax.numpy as jnp
from jax import lax
from jax.experimental import pallas as pl
from jax.experimental.pallas import tpu as pltpu
from jax.experimental.pallas import tpu_sc as plsc

NSC = 2
NTL = 16
NW = NSC * NTL
L = 16
KA = 2000
KC = 2000
SUB = 80
NSUB = KC // SUB
ZR = 160


def _mesh():
    return plsc.VectorSubcoreMesh(core_axis_name="c", subcore_axis_name="s")


def _make_sc_a(E, NP, EPT, NCH):
    EDGES = E
    TW = NP * 8

    @functools.partial(
        pl.kernel,
        out_type=jax.ShapeDtypeStruct((NW * TW,), jnp.float32),
        mesh=_mesh(),
        compiler_params=pltpu.CompilerParams(needs_layout_passes=False),
        scratch_types=[
            pltpu.VMEM((KA,), jnp.int32),
            pltpu.VMEM((KA,), jnp.int32),
            pltpu.VMEM((KA * 4,), jnp.float32),
            pltpu.VMEM((TW,), jnp.float32),
        ],
    )
    def k(src_hbm, dst_hbm, ef_hbm, tab_out, sidx, didx, efb, tab):
        cid = lax.axis_index("c")
        sid = lax.axis_index("s")
        tg = cid * NTL + sid
        iota = lax.iota(jnp.int32, L)
        zero = jnp.zeros((L,), jnp.float32)
        one = jnp.full((L,), 1.0, jnp.float32)
        cidx4 = jnp.where(iota < 4, iota, 0)
        colsel = jnp.where(iota < 5, iota, 5)
        mask5 = iota < 5

        def zl(i, c2):
            tab[pl.ds(i * L, L)] = zero
            return c2

        lax.fori_loop(0, TW // L, zl, 0)

        def chunk(c, carry):
            base = tg * EPT + c * KA
            pltpu.sync_copy(src_hbm.at[pl.ds(base, KA)], sidx)
            pltpu.sync_copy(dst_hbm.at[pl.ds(base, KA)], didx)
            for j in range(4):
                pltpu.sync_copy(ef_hbm.at[pl.ds(j * EDGES + base, KA)],
                                efb.at[pl.ds(j * KA, KA)])

            def dstloop(kk, c2):
                kkv = jnp.full((L,), 0, jnp.int32) + kk
                db = plsc.load_gather(didx, [kkv])
                g = plsc.load_gather(efb, [kkv + cidx4 * KA])
                vals = jnp.where(iota < 4, g, one)
                plsc.addupdate_scatter(tab, [db * 8 + colsel], vals,
                                       mask=mask5)
                return c2

            lax.fori_loop(0, KA, dstloop, 0)

            def srcloop(gi, c2):
                sv = sidx[pl.ds(gi * L, L)]
                idx = sv * 8 + 5
                for l in range(L):
                    plsc.addupdate_scatter(tab, [idx], one,
                                           mask=(iota == l))
                return c2

            lax.fori_loop(0, KA // L, srcloop, 0)

            return carry

        lax.fori_loop(0, NCH, chunk, 0)
        pltpu.sync_copy(tab, tab_out.at[pl.ds(tg * TW, TW)])

    return k


def _make_sc_c(E, N, D, NP, EPT, NCH, RPT):
    nv = D // L

    @functools.partial(
        pl.kernel,
        out_type=jax.ShapeDtypeStruct((NSC * NP, D), jnp.float32),
        mesh=_mesh(),
        compiler_params=pltpu.CompilerParams(needs_layout_passes=False),
        scratch_types=(
            [pltpu.VMEM((SUB,), jnp.int32) for _ in range(2 * NSUB)]
            + [
                pltpu.VMEM((KC,), jnp.float32),
                pltpu.VMEM((SUB, D), jnp.float32),
                pltpu.VMEM((SUB, D), jnp.float32),
                pltpu.SemaphoreType.DMA,
                pltpu.SemaphoreType.DMA,
                pltpu.SemaphoreType.DMA,
                pltpu.VMEM_SHARED((NP, D), jnp.float32),
            ]
        ),
    )
    def k(src_hbm, dst_hbm, att_hbm, fs_hbm, zeros_hbm, h_out, *scr):
        sidxs = scr[:NSUB]
        didxs = scr[NSUB:2 * NSUB]
        attb, rows0, rows1, isem, sem0, sem1, acc = scr[2 * NSUB:]
        rows = (rows0, rows1)
        sems = (sem0, sem1)
        cid = lax.axis_index("c")
        sid = lax.axis_index("s")
        tg = cid * NTL + sid
        r0 = sid * RPT
        off = 0
        rem = RPT
        while rem > 0:
            sz = ZR if rem >= ZR else rem
            pltpu.sync_copy(zeros_hbm.at[pl.ds(0, sz)],
                            acc.at[pl.ds(r0 + off, sz)])
            off += sz
            rem -= sz
        plsc.subcore_barrier()

        def block(blk, carry):
            ebase = tg * EPT + blk * KC
            cps = []
            for c in range(NSUB):
                cps.append(pltpu.async_copy(
                    src_hbm.at[pl.ds(ebase + c * SUB, SUB)], sidxs[c], isem))
                cps.append(pltpu.async_copy(
                    dst_hbm.at[pl.ds(ebase + c * SUB, SUB)], didxs[c], isem))
            cps.append(pltpu.async_copy(
                att_hbm.at[pl.ds(ebase, KC)], attb, isem))
            for cp in cps:
                cp.wait()

            gq = []
            gq.append(pltpu.async_copy(fs_hbm.at[sidxs[0]], rows[0], sems[0]))
            for c in range(NSUB):
                b = c % 2
                if c + 1 < NSUB:
                    gq.append(pltpu.async_copy(
                        fs_hbm.at[sidxs[c + 1]], rows[1 - b], sems[1 - b]))
                gq[c].wait()

                def scale(kk, c2, _b=b, _c=c):
                    a = plsc.load_gather(
                        attb, [jnp.full((L,), _c * SUB, jnp.int32) + kk])
                    for j in range(nv):
                        rows[_b][kk, pl.ds(j * L, L)] = (
                            rows[_b][kk, pl.ds(j * L, L)] * a)
                    return c2

                lax.fori_loop(0, SUB, scale, 0)

                pltpu.sync_copy(rows[b], acc.at[didxs[c]], add=True)
            return carry

        lax.fori_loop(0, NCH, block, 0)
        plsc.subcore_barrier()
        pltpu.sync_copy(acc.at[pl.ds(r0, RPT)],
                        h_out.at[pl.ds(cid * NP + r0, RPT)])

    return k


def _make_tc_b1(N, D, NP):
    BR1 = 128

    def body(feats_ref, tab_ref, fs_ref, red_ref):
        red = jnp.sum(tab_ref[...], axis=0).reshape(BR1, 8)
        deg = red[:, 5]
        dis = jnp.where(deg > 0, lax.rsqrt(deg), 0.0)
        fs_ref[...] = feats_ref[...] * dis[:, None]
        red_ref[...] = red

    return pl.pallas_call(
        body,
        grid=(NP // BR1,),
        in_specs=[
            pl.BlockSpec((BR1, D), lambda i: (i, 0)),
            pl.BlockSpec((NW, BR1 * 8 // 128, 128), lambda i: (0, i, 0)),
        ],
        out_specs=[
            pl.BlockSpec((BR1, D), lambda i: (i, 0)),
            pl.BlockSpec((BR1, 8), lambda i: (i, 0)),
        ],
        out_shape=[
            jax.ShapeDtypeStruct((N, D), jnp.float32),
            jax.ShapeDtypeStruct((N, 8), jnp.float32),
        ],
    )


def _make_tc_b2(E, NB2, BR2):
    def body(ef_ref, w_ref, b_ref, att_ref, eff_ref):
        i = pl.program_id(0)
        ef = ef_ref[...]
        z = (ef[:, 0] * w_ref[0, 0] + ef[:, 1] * w_ref[0, 1]
             + ef[:, 2] * w_ref[0, 2] + ef[:, 3] * w_ref[0, 3] + b_ref[0, 0])
        att_ref[pl.ds(i * BR2, BR2)] = 1.0 / (1.0 + jnp.exp(-z))
        for j in range(4):
            eff_ref[pl.ds(j * E + i * BR2, BR2)] = ef[:, j]

    return pl.pallas_call(
        body,
        grid=(NB2,),
        in_specs=[
            pl.BlockSpec((BR2, 4), lambda i: (i, 0)),
            pl.BlockSpec((1, 4), lambda i: (0, 0)),
            pl.BlockSpec((1, 1), lambda i: (0, 0)),
        ],
        out_specs=[
            pl.BlockSpec((E,), lambda i: (0,)),
            pl.BlockSpec((4 * E,), lambda i: (0,)),
        ],
        out_shape=[
            jax.ShapeDtypeStruct((E,), jnp.float32),
            jax.ShapeDtypeStruct((4 * E,), jnp.float32),
        ],
    )


def _make_tc_d(N, D, NB, BR):
    def body(hp_ref, red_ref, feats_ref, fs_ref,
             wet_ref, wmt_ref, wst_ref, be_ref, bm_ref, bs_ref, out_ref):
        f32 = jnp.float32
        h = hp_ref[0] + hp_ref[1]
        red = red_ref[...]
        sef = red[:, 0:4]
        indeg = red[:, 4:5]
        deg = red[:, 5]
        dis = jnp.where(deg > 0, lax.rsqrt(deg), 0.0)[:, None]
        rst = (h + jnp.dot(sef, wet_ref[...], preferred_element_type=f32)
               + indeg * be_ref[...] + indeg * fs_ref[...])
        out_ref[...] = (
            jnp.dot(rst * dis, wmt_ref[...], preferred_element_type=f32)
            + bm_ref[...]
            + jnp.dot(feats_ref[...], wst_ref[...], preferred_element_type=f32)
            + bs_ref[...])

    return pl.pallas_call(
        body,
        grid=(NB,),
        in_specs=[
            pl.BlockSpec((NSC, BR, D), lambda i: (0, i, 0)),
            pl.BlockSpec((BR, 8), lambda i: (i, 0)),
            pl.BlockSpec((BR, D), lambda i: (i, 0)),
            pl.BlockSpec((BR, D), lambda i: (i, 0)),
            pl.BlockSpec((4, D), lambda i: (0, 0)),
            pl.BlockSpec((D, D), lambda i: (0, 0)),
            pl.BlockSpec((D, D), lambda i: (0, 0)),
            pl.BlockSpec((1, D), lambda i: (0, 0)),
            pl.BlockSpec((1, D), lambda i: (0, 0)),
            pl.BlockSpec((1, D), lambda i: (0, 0)),
        ],
        out_specs=pl.BlockSpec((BR, D), lambda i: (i, 0)),
        out_shape=jax.ShapeDtypeStruct((N, D), jnp.float32),
    )


def kernel(feats, edge_index, edge_feats, linear_skip_weight, linear_skip_bias,
           linear_msg_weight, linear_msg_bias, W_edge, b_edge, W_att, b_att):
    N, D = feats.shape
    E = edge_index.shape[1]
    NP = ((N + 127) // 128) * 128
    RPT = NP // NTL
    EPT = E // NW
    NCHA = EPT // KA
    NCHC = EPT // KC
    BR = 400
    NB = N // BR
    src = edge_index[0]
    dst = edge_index[1]
    zeros_c = jnp.zeros((ZR, D), jnp.float32)

    att, eff = _make_tc_b2(E, E // 6400, 6400)(
        edge_feats, W_att, b_att.reshape(1, 1))
    tabs = _make_sc_a(E, NP, EPT, NCHA)(src, dst, eff)
    fs, red = _make_tc_b1(N, D, NP)(
        feats, tabs.reshape(NW, NP * 8 // 128, 128))

    hp = _make_sc_c(E, N, D, NP, EPT, NCHC, RPT)(src, dst, att, fs, zeros_c)

    out = _make_tc_d(N, D, NB, BR)(
        hp.reshape(NSC, NP, D), red, feats, fs,
        W_edge.T, linear_msg_weight.T, linear_skip_weight.T,
        b_edge.reshape(1, D), linear_msg_bias.reshape(1, D),
        linear_skip_bias.reshape(1, D))
    return out


# speedup vs baseline: 1.7667x; 1.7667x over previous
import functools

import jax
import jax.numpy as jnp
from jax import lax
from jax.experimental import pallas as pl
from jax.experimental.pallas import tpu as pltpu
from jax.experimental.pallas import tpu_sc as plsc

NSC = 2
NTL = 16
NW = NSC * NTL
L = 16
KA = 2000
KC = 2000
SUB = 80
NSUB = KC // SUB
ZR = 160


def _mesh():
    return plsc.VectorSubcoreMesh(core_axis_name="c", subcore_axis_name="s")


def _make_sc_a(E, NP, EPT, NCH):
    TW = NP * 8

    @functools.partial(
        pl.kernel,
        out_type=jax.ShapeDtypeStruct((NW * TW,), jnp.float32),
        mesh=_mesh(),
        compiler_params=pltpu.CompilerParams(needs_layout_passes=False),
        scratch_types=[
            pltpu.VMEM((KA,), jnp.int32),
            pltpu.VMEM((KA,), jnp.int32),
            pltpu.VMEM((KA * 4,), jnp.float32),
            pltpu.VMEM((TW,), jnp.float32),
        ],
    )
    def k(src_hbm, dst_hbm, ef_hbm, tab_out, sidx, didx, efb, tab):
        cid = lax.axis_index("c")
        sid = lax.axis_index("s")
        tg = cid * NTL + sid
        iota = lax.iota(jnp.int32, L)
        zero = jnp.zeros((L,), jnp.float32)
        one = jnp.full((L,), 1.0, jnp.float32)
        cidx4 = jnp.where(iota < 4, iota, 0)
        colsel = jnp.where(iota < 5, iota, 5)
        mask5 = iota < 5

        def zl(i, c2):
            tab[pl.ds(i * L, L)] = zero
            return c2

        lax.fori_loop(0, TW // L, zl, 0)

        def chunk(c, carry):
            base = tg * EPT + c * KA
            pltpu.sync_copy(src_hbm.at[pl.ds(base, KA)], sidx)
            pltpu.sync_copy(dst_hbm.at[pl.ds(base, KA)], didx)
            pltpu.sync_copy(ef_hbm.at[pl.ds(base * 4, KA * 4)], efb)

            def dstloop(kk, c2):
                kkv = jnp.full((L,), 0, jnp.int32) + kk
                db = plsc.load_gather(didx, [kkv])
                g = plsc.load_gather(efb, [kkv * 4 + cidx4])
                vals = jnp.where(iota < 4, g, one)
                plsc.addupdate_scatter(tab, [db * 8 + colsel], vals,
                                       mask=mask5)
                return c2

            lax.fori_loop(0, KA, dstloop, 0)

            def srcloop(gi, c2):
                sv = sidx[pl.ds(gi * L, L)]
                idx = sv * 8 + 5
                for l in range(L):
                    plsc.addupdate_scatter(tab, [idx], one,
                                           mask=(iota == l))
                return c2

            lax.fori_loop(0, KA // L, srcloop, 0)

            return carry

        lax.fori_loop(0, NCH, chunk, 0)
        pltpu.sync_copy(tab, tab_out.at[pl.ds(tg * TW, TW)])

    return k


def _make_sc_c(E, N, D, NP, EPT, NCH, RPT):
    nv = D // L

    @functools.partial(
        pl.kernel,
        out_type=jax.ShapeDtypeStruct((NSC * NP, D), jnp.float32),
        mesh=_mesh(),
        compiler_params=pltpu.CompilerParams(needs_layout_passes=False),
        scratch_types=(
            [pltpu.VMEM((SUB,), jnp.int32) for _ in range(2 * NSUB)]
            + [
                pltpu.VMEM((KC,), jnp.float32),
                pltpu.VMEM((SUB, D), jnp.float32),
                pltpu.VMEM((SUB, D), jnp.float32),
                pltpu.SemaphoreType.DMA,
                pltpu.SemaphoreType.DMA,
                pltpu.SemaphoreType.DMA,
                pltpu.VMEM_SHARED((NP, D), jnp.float32),
            ]
        ),
    )
    def k(src_hbm, dst_hbm, att_hbm, fs_hbm, zeros_hbm, h_out, *scr):
        sidxs = scr[:NSUB]
        didxs = scr[NSUB:2 * NSUB]
        attb, rows0, rows1, isem, sem0, sem1, acc = scr[2 * NSUB:]
        rows = (rows0, rows1)
        sems = (sem0, sem1)
        cid = lax.axis_index("c")
        sid = lax.axis_index("s")
        tg = cid * NTL + sid
        r0 = sid * RPT
        off = 0
        rem = RPT
        while rem > 0:
            sz = ZR if rem >= ZR else rem
            pltpu.sync_copy(zeros_hbm.at[pl.ds(0, sz)],
                            acc.at[pl.ds(r0 + off, sz)])
            off += sz
            rem -= sz
        plsc.subcore_barrier()

        def block(blk, carry):
            ebase = tg * EPT + blk * KC
            cps = []
            for c in range(NSUB):
                cps.append(pltpu.async_copy(
                    src_hbm.at[pl.ds(ebase + c * SUB, SUB)], sidxs[c], isem))
                cps.append(pltpu.async_copy(
                    dst_hbm.at[pl.ds(ebase + c * SUB, SUB)], didxs[c], isem))
            cps.append(pltpu.async_copy(
                att_hbm.at[pl.ds(ebase, KC)], attb, isem))
            for cp in cps:
                cp.wait()

            gq = []
            gq.append(pltpu.async_copy(fs_hbm.at[sidxs[0]], rows[0], sems[0]))
            for c in range(NSUB):
                b = c % 2
                if c + 1 < NSUB:
                    gq.append(pltpu.async_copy(
                        fs_hbm.at[sidxs[c + 1]], rows[1 - b], sems[1 - b]))
                gq[c].wait()

                def scale(kk, c2, _b=b, _c=c):
                    a = plsc.load_gather(
                        attb, [jnp.full((L,), _c * SUB, jnp.int32) + kk])
                    for j in range(nv):
                        rows[_b][kk, pl.ds(j * L, L)] = (
                            rows[_b][kk, pl.ds(j * L, L)] * a)
                    return c2

                lax.fori_loop(0, SUB, scale, 0)

                pltpu.sync_copy(rows[b], acc.at[didxs[c]], add=True)
            return carry

        lax.fori_loop(0, NCH, block, 0)
        plsc.subcore_barrier()
        pltpu.sync_copy(acc.at[pl.ds(r0, RPT)],
                        h_out.at[pl.ds(cid * NP + r0, RPT)])

    return k


def _make_tc_b1(N, D, NP):
    BR1 = 128

    def body(feats_ref, tab_ref, fs_ref, red_ref):
        red = jnp.sum(tab_ref[...], axis=0).reshape(BR1, 8)
        deg = red[:, 5]
        dis = jnp.where(deg > 0, lax.rsqrt(deg), 0.0)
        fs_ref[...] = feats_ref[...] * dis[:, None]
        red_ref[...] = red

    return pl.pallas_call(
        body,
        grid=(NP // BR1,),
        in_specs=[
            pl.BlockSpec((BR1, D), lambda i: (i, 0)),
            pl.BlockSpec((NW, BR1 * 8 // 128, 128), lambda i: (0, i, 0)),
        ],
        out_specs=[
            pl.BlockSpec((BR1, D), lambda i: (i, 0)),
            pl.BlockSpec((BR1, 8), lambda i: (i, 0)),
        ],
        out_shape=[
            jax.ShapeDtypeStruct((N, D), jnp.float32),
            jax.ShapeDtypeStruct((N, 8), jnp.float32),
        ],
    )


def _make_tc_b2(ER, NBE, BRE):
    def body(ef_ref, wpat_ref, sel_ref, b_ref, att_ref):
        z = ef_ref[...] * wpat_ref[...]
        z32 = jnp.dot(z, sel_ref[...], preferred_element_type=jnp.float32)
        att_ref[...] = 1.0 / (1.0 + jnp.exp(-(z32 + b_ref[0, 0])))

    return pl.pallas_call(
        body,
        grid=(NBE,),
        in_specs=[
            pl.BlockSpec((BRE, 128), lambda i: (i, 0)),
            pl.BlockSpec((1, 128), lambda i: (0, 0)),
            pl.BlockSpec((128, 32), lambda i: (0, 0)),
            pl.BlockSpec((1, 128), lambda i: (0, 0)),
        ],
        out_specs=pl.BlockSpec((BRE, 32), lambda i: (i, 0)),
        out_shape=jax.ShapeDtypeStruct((ER, 32), jnp.float32),
    )


def _make_tc_d(N, D, NB, BR):
    def body(hp_ref, red_ref, feats_ref, fs_ref,
             wet_ref, wmt_ref, wst_ref, be_ref, bm_ref, bs_ref, out_ref):
        f32 = jnp.float32
        h = hp_ref[0] + hp_ref[1]
        red = red_ref[...]
        sef = red[:, 0:4]
        indeg = red[:, 4:5]
        deg = red[:, 5]
        dis = jnp.where(deg > 0, lax.rsqrt(deg), 0.0)[:, None]
        rst = (h + jnp.dot(sef, wet_ref[...], preferred_element_type=f32)
               + indeg * be_ref[...] + indeg * fs_ref[...])
        out_ref[...] = (
            jnp.dot(rst * dis, wmt_ref[...], preferred_element_type=f32)
            + bm_ref[...]
            + jnp.dot(feats_ref[...], wst_ref[...], preferred_element_type=f32)
            + bs_ref[...])

    return pl.pallas_call(
        body,
        grid=(NB,),
        in_specs=[
            pl.BlockSpec((NSC, BR, D), lambda i: (0, i, 0)),
            pl.BlockSpec((BR, 8), lambda i: (i, 0)),
            pl.BlockSpec((BR, D), lambda i: (i, 0)),
            pl.BlockSpec((BR, D), lambda i: (i, 0)),
            pl.BlockSpec((4, D), lambda i: (0, 0)),
            pl.BlockSpec((D, D), lambda i: (0, 0)),
            pl.BlockSpec((D, D), lambda i: (0, 0)),
            pl.BlockSpec((1, D), lambda i: (0, 0)),
            pl.BlockSpec((1, D), lambda i: (0, 0)),
            pl.BlockSpec((1, D), lambda i: (0, 0)),
        ],
        out_specs=pl.BlockSpec((BR, D), lambda i: (i, 0)),
        out_shape=jax.ShapeDtypeStruct((N, D), jnp.float32),
    )


def kernel(feats, edge_index, edge_feats, linear_skip_weight, linear_skip_bias,
           linear_msg_weight, linear_msg_bias, W_edge, b_edge, W_att, b_att):
    N, D = feats.shape
    E = edge_index.shape[1]
    NP = ((N + 127) // 128) * 128
    RPT = NP // NTL
    EPT = E // NW
    NCHA = EPT // KA
    NCHC = EPT // KC
    BR = 400
    NB = N // BR
    src = edge_index[0]
    dst = edge_index[1]
    zeros_c = jnp.zeros((ZR, D), jnp.float32)

    ER = E // 32
    eff = edge_feats.reshape(E * 4)
    ef2 = eff.reshape(ER, 128)
    wpat = jnp.tile(W_att[0], 32).reshape(1, 128)
    sel = (jnp.arange(128)[:, None] // 4 == jnp.arange(32)[None, :]
           ).astype(jnp.float32)
    b128 = jnp.tile(b_att, 128).reshape(1, 128)
    att = _make_tc_b2(ER, ER // 400, 400)(ef2, wpat, sel, b128).reshape(E)
    tabs = _make_sc_a(E, NP, EPT, NCHA)(src, dst, eff)
    fs, red = _make_tc_b1(N, D, NP)(
        feats, tabs.reshape(NW, NP * 8 // 128, 128))

    hp = _make_sc_c(E, N, D, NP, EPT, NCHC, RPT)(src, dst, att, fs, zeros_c)

    out = _make_tc_d(N, D, NB, BR)(
        hp.reshape(NSC, NP, D), red, feats, fs,
        W_edge.T, linear_msg_weight.T, linear_skip_weight.T,
        b_edge.reshape(1, D), linear_msg_bias.reshape(1, D),
        linear_skip_bias.reshape(1, D))
    return out
